# baseline (speedup 1.0000x reference)
import jax
import jax.numpy as jnp
from jax import lax
from jax.experimental import pallas as pl
from jax.experimental.pallas import tpu as pltpu
from jax.experimental.pallas import tpu_sc as plsc

BATCH = 4096
WIDTH = 26
WPAD = 32
DIM = 64
TOTAL = BATCH * WIDTH
BPC = 4
CHUNK = BPC * WIDTH
NBUF = 4

_info = plsc.get_sparse_core_info()
NC, NS = _info.num_cores, _info.num_subcores
NW = NC * NS
PER_W = TOTAL // NW
NSTEP = PER_W // CHUNK
B_PER_W = BATCH // NW
assert PER_W * NW == TOTAL and NSTEP * CHUNK == PER_W


def _gather_body(idx_hbm, table_hbm, out_hbm, idxs, rows, gsems, wsems):
    wid = lax.axis_index("s") * NC + lax.axis_index("c")
    base = wid * PER_W
    bbase = wid * B_PER_W

    def gather(s):
        b = s % NBUF
        pltpu.sync_copy(idx_hbm.at[pl.ds(base + s * CHUNK, CHUNK)], idxs[b])
        pltpu.async_copy(table_hbm.at[idxs[b]], rows[b], gsems[b])

    def zero_fixup(b):
        def group(g, carry):
            start = jnp.minimum(g * 16, CHUNK - 16)
            iv = idxs[b][pl.ds(start, 16)]
            m = iv == 0
            nz = jnp.max(plsc.all_reduce_population_count(m))

            @pl.when(nz > 0)
            def _():
                rid = start + jnp.arange(16, dtype=jnp.int32)
                zeros = jnp.zeros((16,), jnp.float32)

                def dcol(d, c):
                    cid = jnp.full((16,), d, jnp.int32)
                    plsc.store_scatter(rows[b], [rid, cid], zeros, mask=m)
                    return c

                lax.fori_loop(0, DIM, dcol, 0)

            return carry

        lax.fori_loop(0, (CHUNK + 15) // 16, group, 0)

    def writes(s):
        b = s % NBUF
        for j in range(BPC):
            pltpu.async_copy(
                rows[b].at[pl.ds(j * WIDTH, WIDTH)],
                out_hbm.at[bbase + s * BPC + j, pl.ds(0, WIDTH)],
                wsems[b],
            )

    def wait_writes(s):
        b = s % NBUF
        for j in range(BPC):
            pltpu.make_async_copy(
                rows[b].at[pl.ds(j * WIDTH, WIDTH)],
                out_hbm.at[bbase + s * BPC + j, pl.ds(0, WIDTH)],
                wsems[b],
            ).wait()

    for s in range(NBUF):
        gather(s)

    for s in range(NSTEP):
        b = s % NBUF
        pltpu.make_async_copy(table_hbm.at[idxs[b]], rows[b], gsems[b]).wait()
        zero_fixup(b)
        writes(s)
        wait_writes(s)
        if s + NBUF < NSTEP:
            gather(s + NBUF)


def _gather(idx_flat, table):
    mesh = plsc.VectorSubcoreMesh(core_axis_name="c", subcore_axis_name="s")
    scratch = [
        [pltpu.VMEM((CHUNK,), jnp.int32) for _ in range(NBUF)],
        [pltpu.VMEM((CHUNK, DIM), jnp.float32) for _ in range(NBUF)],
        [pltpu.SemaphoreType.DMA for _ in range(NBUF)],
        [pltpu.SemaphoreType.DMA for _ in range(NBUF)],
    ]
    k = pl.kernel(
        _gather_body,
        mesh=mesh,
        out_type=jax.ShapeDtypeStruct((BATCH, WPAD, DIM), jnp.float32),
        scratch_types=scratch,
        compiler_params=pltpu.CompilerParams(
            use_tc_tiling_on_sc=False, needs_layout_passes=False
        ),
    )
    return k(idx_flat, table)


_B_BLK = 256


def _finish_body(rows_ref, out_ref):
    out_ref[...] = rows_ref[...][:, :WIDTH, :]


def _finish(rows):
    return pl.pallas_call(
        _finish_body,
        grid=(BATCH // _B_BLK,),
        in_specs=[
            pl.BlockSpec((_B_BLK, WPAD, DIM), lambda i: (i, 0, 0)),
        ],
        out_specs=pl.BlockSpec((_B_BLK, WIDTH, DIM), lambda i: (i, 0, 0)),
        out_shape=jax.ShapeDtypeStruct((BATCH, WIDTH, DIM), jnp.float32),
    )(rows)


@jax.jit
def _run(idx, table):
    idx_flat = idx.reshape(TOTAL)
    rows = _gather(idx_flat, table)
    return _finish(rows)


def kernel(input, table):
    return _run(input.astype(jnp.int32), table)


# speedup vs baseline: 1.0013x; 1.0013x over previous
import jax
import jax.numpy as jnp
from jax import lax
from jax.experimental import pallas as pl
from jax.experimental.pallas import tpu as pltpu
from jax.experimental.pallas import tpu_sc as plsc

BATCH = 4096
WIDTH = 26
WPAD = 32
DIM = 64
TOTAL = BATCH * WIDTH
BPC = 4
CHUNK = BPC * WIDTH
NBUF = 4

_info = plsc.get_sparse_core_info()
NC, NS = _info.num_cores, _info.num_subcores
NW = NC * NS
PER_W = TOTAL // NW
NSTEP = PER_W // CHUNK
B_PER_W = BATCH // NW
assert PER_W * NW == TOTAL and NSTEP * CHUNK == PER_W


def _gather_body(idx_hbm, table_hbm, out_hbm, idxs, rows, gsems, wsems):
    wid = lax.axis_index("s") * NC + lax.axis_index("c")
    base = wid * PER_W
    bbase = wid * B_PER_W

    def gather(s):
        b = s % NBUF
        pltpu.sync_copy(idx_hbm.at[pl.ds(base + s * CHUNK, CHUNK)], idxs[b])
        pltpu.async_copy(table_hbm.at[idxs[b]], rows[b], gsems[b])

    def zero_fixup(b):
        def group(g, carry):
            start = jnp.minimum(g * 16, CHUNK - 16)
            iv = idxs[b][pl.ds(start, 16)]
            m = iv == 0
            nz = jnp.max(plsc.all_reduce_population_count(m))

            @pl.when(nz > 0)
            def _():
                rid = start + jnp.arange(16, dtype=jnp.int32)
                zeros = jnp.zeros((16,), jnp.float32)

                def dcol(d, c):
                    cid = jnp.full((16,), d, jnp.int32)
                    plsc.store_scatter(rows[b], [rid, cid], zeros, mask=m)
                    return c

                lax.fori_loop(0, DIM, dcol, 0)

            return carry

        lax.fori_loop(0, (CHUNK + 15) // 16, group, 0)

    def writes(s):
        b = s % NBUF
        for j in range(BPC):
            pltpu.async_copy(
                rows[b].at[pl.ds(j * WIDTH, WIDTH)],
                out_hbm.at[pl.ds((bbase + s * BPC + j) * WPAD, WIDTH)],
                wsems[b],
            )

    def wait_writes(s):
        b = s % NBUF
        for j in range(BPC):
            pltpu.make_async_copy(
                rows[b].at[pl.ds(j * WIDTH, WIDTH)],
                out_hbm.at[pl.ds((bbase + s * BPC + j) * WPAD, WIDTH)],
                wsems[b],
            ).wait()

    for s in range(NBUF):
        gather(s)

    for s in range(NSTEP):
        b = s % NBUF
        pltpu.make_async_copy(table_hbm.at[idxs[b]], rows[b], gsems[b]).wait()
        zero_fixup(b)
        writes(s)
        wait_writes(s)
        if s + NBUF < NSTEP:
            gather(s + NBUF)


def _gather(idx_flat, table):
    mesh = plsc.VectorSubcoreMesh(core_axis_name="c", subcore_axis_name="s")
    scratch = [
        [pltpu.VMEM((CHUNK,), jnp.int32) for _ in range(NBUF)],
        [pltpu.VMEM((CHUNK, DIM), jnp.float32) for _ in range(NBUF)],
        [pltpu.SemaphoreType.DMA for _ in range(NBUF)],
        [pltpu.SemaphoreType.DMA for _ in range(NBUF)],
    ]
    k = pl.kernel(
        _gather_body,
        mesh=mesh,
        out_type=jax.ShapeDtypeStruct((BATCH * WPAD, DIM), jnp.float32),
        scratch_types=scratch,
        compiler_params=pltpu.CompilerParams(
            use_tc_tiling_on_sc=False, needs_layout_passes=False
        ),
    )
    return k(idx_flat, table)


_B_BLK = 256


def _finish_body(rows_ref, out_ref):
    out_ref[...] = rows_ref[...][:, :WIDTH, :]


def _finish(rows):
    return pl.pallas_call(
        _finish_body,
        grid=(BATCH // _B_BLK,),
        in_specs=[
            pl.BlockSpec((_B_BLK, WPAD, DIM), lambda i: (i, 0, 0)),
        ],
        out_specs=pl.BlockSpec((_B_BLK, WIDTH, DIM), lambda i: (i, 0, 0)),
        out_shape=jax.ShapeDtypeStruct((BATCH, WIDTH, DIM), jnp.float32),
    )(rows)


@jax.jit
def _run(idx, table):
    idx_flat = idx.reshape(TOTAL)
    rows = _gather(idx_flat, table)
    return _finish(rows.reshape(BATCH, WPAD, DIM))


def kernel(input, table):
    return _run(input.astype(jnp.int32), table)
